# Initial kernel scaffold; baseline (speedup 1.0000x reference)
import jax
import jax.numpy as jnp
from jax.experimental import pallas as pl


def kernel(adj_u1_indices, adj_u1_values, adj_i1_indices, adj_i1_values, user_emb, item_emb, W_u, W_i, attn_u_w, attn_u_b, attn_i_w, attn_i_b):
    raise NotImplementedError("write your pallas kernel here")


# speedup vs baseline: 8.8072x; 8.8072x over previous
import jax
import jax.numpy as jnp
from jax import lax
from jax.experimental import pallas as pl
from jax.experimental.pallas import tpu as pltpu
from jax.experimental.pallas import tpu_sc as plsc

N = 100000
D = 32
H = 16
E = 1600000
NCORES = 2
NTILES = 16
SUB = 128
NSUB = 8
C = SUB * NSUB
N_PAD = 100352
E_PAD = 1605632


def _lane_splat(v16, k):
  idx = jnp.full((16, 1), k, jnp.int32)
  dnums = lax.GatherDimensionNumbers(
      offset_dims=(), collapsed_slice_dims=(0,), start_index_map=(0,))
  return lax.gather(v16, idx, dnums, (1,),
                    mode=lax.GatherScatterMode.PROMISE_IN_BOUNDS)


def _make_spmm():
  e_t = E_PAD // NTILES
  nch = e_t // C
  r_t = e_t // SUB
  nz = N_PAD // NTILES
  zc = 896
  nzrep = nz // zc

  mesh = plsc.VectorSubcoreMesh(
      core_axis_name="c", subcore_axis_name="s",
      num_cores=NCORES, num_subcores=NTILES)

  def body(cols_ref, rows_ref, vals_ref, table_ref, out_ref,
           acc, idx_v, row_v, val_v, gath_v, gsem):
    c = lax.axis_index("c")
    s = lax.axis_index("s")

    def zb(j, carry):
      gath_v[j] = jnp.zeros((H,), jnp.float32)
      return carry
    lax.fori_loop(0, zc, zb, 0)
    for r in range(nzrep):
      pltpu.sync_copy(gath_v.at[pl.ds(0, zc)],
                      acc.at[pl.ds(s * nz + r * zc, zc)])
    plsc.subcore_barrier()

    def chunk(g, carry):
      rb = s * r_t + g * NSUB
      pltpu.sync_copy(cols_ref.at[c, pl.ds(rb, NSUB)], idx_v)
      pltpu.sync_copy(rows_ref.at[pl.ds(rb, NSUB)], row_v)
      eb = s * e_t + g * C
      pltpu.sync_copy(vals_ref.at[pl.ds(eb, C)], val_v)
      descs = [
          pltpu.async_copy(table_ref.at[idx_v.at[j]],
                           gath_v.at[pl.ds(j * SUB, SUB)], gsem)
          for j in range(NSUB)
      ]
      for d in descs:
        d.wait()

      def grp(gg, carry2):
        v16 = val_v[pl.ds(gg * 16, 16)]
        for k in range(16):
          j = gg * 16 + k
          vv = _lane_splat(v16, k)
          gath_v[j] = gath_v[j] * vv
        return carry2
      lax.fori_loop(0, C // 16, grp, 0)

      for j in range(NSUB):
        pltpu.sync_copy(gath_v.at[pl.ds(j * SUB, SUB)],
                        acc.at[row_v.at[j]], add=True)
      return carry
    lax.fori_loop(0, nch, chunk, 0)

    plsc.subcore_barrier()
    pltpu.sync_copy(acc.at[pl.ds(s * nz, nz)],
                    out_ref.at[pl.ds(c * N_PAD + s * nz, nz)])

  return pl.kernel(
      body,
      out_type=jax.ShapeDtypeStruct((2 * N_PAD, H), jnp.float32),
      mesh=mesh,
      compiler_params=pltpu.CompilerParams(use_tc_tiling_on_sc=False),
      scratch_types=[
          pltpu.VMEM_SHARED((N_PAD, H), jnp.float32),
          pltpu.VMEM((NSUB, SUB), jnp.int32),
          pltpu.VMEM((NSUB, SUB), jnp.int32),
          pltpu.VMEM((C,), jnp.float32),
          pltpu.VMEM((C, H), jnp.float32),
          pltpu.SemaphoreType.DMA,
      ],
  )


_BM = 2000


def _mm_body(s_ref, w_ref, hstd_ref, hsp_ref):
  w = w_ref[...]
  s0 = s_ref[0]
  s1 = s_ref[1]
  x = (jnp.dot(s0, w[:H, :], preferred_element_type=jnp.float32) +
       jnp.dot(s1, w[H:, :], preferred_element_type=jnp.float32))
  h = jnp.maximum(x, 0.0)
  hstd_ref[...] = h
  hsp_ref[0] = h[:, :H]
  hsp_ref[1] = h[:, H:]


def _matmul_relu(s2, w):
  grid = N // _BM
  return pl.pallas_call(
      _mm_body,
      grid=(grid,),
      in_specs=[
          pl.BlockSpec((2, _BM, H), lambda i: (0, i, 0)),
          pl.BlockSpec((D, D), lambda i: (0, 0)),
      ],
      out_specs=[
          pl.BlockSpec((_BM, D), lambda i: (i, 0)),
          pl.BlockSpec((2, _BM, H), lambda i: (0, i, 0)),
      ],
      out_shape=[
          jax.ShapeDtypeStruct((N, D), jnp.float32),
          jax.ShapeDtypeStruct((2, N_PAD, H), jnp.float32),
      ],
  )(s2, w)


def _pool_body(e0_ref, e1_ref, e2_ref, w_ref, b_ref, out_ref):
  w = w_ref[...]
  b = b_ref[0, 0]
  e0 = e0_ref[...]
  e1 = e1_ref[...]
  e2 = e2_ref[...]
  a0 = jax.nn.sigmoid(jnp.sum(e0 * w, axis=1, keepdims=True) + b)
  a1 = jax.nn.sigmoid(jnp.sum(e1 * w, axis=1, keepdims=True) + b)
  a2 = jax.nn.sigmoid(jnp.sum(e2 * w, axis=1, keepdims=True) + b)
  m = jnp.maximum(jnp.maximum(a0, a1), a2)
  x0 = jnp.exp(a0 - m)
  x1 = jnp.exp(a1 - m)
  x2 = jnp.exp(a2 - m)
  inv = 1.0 / (x0 + x1 + x2)
  out_ref[...] = (e0 * x0 + e1 * x1 + e2 * x2) * inv


def _pool(e0, e1, e2, w_row, b11):
  grid = N // _BM
  blk = pl.BlockSpec((_BM, D), lambda i: (i, 0))
  return pl.pallas_call(
      _pool_body,
      grid=(grid,),
      in_specs=[blk, blk, blk,
                pl.BlockSpec((1, D), lambda i: (0, 0)),
                pl.BlockSpec((1, 1), lambda i: (0, 0))],
      out_specs=blk,
      out_shape=jax.ShapeDtypeStruct((N, D), jnp.float32),
  )(e0, e1, e2, w_row, b11)


def _split_pad(x):
  x2 = x.reshape(N, 2, H).transpose(1, 0, 2)
  x2 = jnp.pad(x2, ((0, 0), (0, N_PAD - N), (0, 0)))
  return x2.reshape(2 * N_PAD, H)


def _branch(indices, values, emb0, w, attn_w, attn_b, spmm):
  rows = jnp.pad(indices[0], (0, E_PAD - E))
  cols = jnp.pad(indices[1], (0, E_PAD - E))
  vals = jnp.pad(values, (0, E_PAD - E))
  cols3 = jnp.stack([cols, cols + N_PAD]).reshape(2, E_PAD // SUB, SUB)
  rows3 = rows.reshape(E_PAD // SUB, SUB)

  s0 = spmm(cols3, rows3, vals, _split_pad(emb0))
  h1_std, h1_sp = _matmul_relu(s0.reshape(2, N_PAD, H), w)
  s1 = spmm(cols3, rows3, vals, h1_sp.reshape(2 * N_PAD, H))
  h2_std, _ = _matmul_relu(s1.reshape(2, N_PAD, H), w)
  return _pool(emb0, h1_std, h2_std,
               attn_w.reshape(1, D), attn_b.reshape(1, 1))


def kernel(adj_u1_indices, adj_u1_values, adj_i1_indices, adj_i1_values,
           user_emb, item_emb, W_u, W_i,
           attn_u_w, attn_u_b, attn_i_w, attn_i_b):
  spmm = _make_spmm()
  u_out = _branch(adj_u1_indices, adj_u1_values, user_emb, W_u,
                  attn_u_w, attn_u_b, spmm)
  i_out = _branch(adj_i1_indices, adj_i1_values, item_emb, W_i,
                  attn_i_w, attn_i_b, spmm)
  return (u_out, i_out)


# speedup vs baseline: 12.8260x; 1.4563x over previous
import jax
import jax.numpy as jnp
from jax import lax
from jax.experimental import pallas as pl
from jax.experimental.pallas import tpu as pltpu
from jax.experimental.pallas import tpu_sc as plsc

N = 100000
D = 32
H = 16
E = 1600000
NCORES = 2
NTILES = 16
SUB = 128
NSUB = 4
C = SUB * NSUB
N_PAD = 100352
E_PAD = 1605632


def _lane_splat(v16, k):
  idx = jnp.full((16, 1), k, jnp.int32)
  dnums = lax.GatherDimensionNumbers(
      offset_dims=(), collapsed_slice_dims=(0,), start_index_map=(0,))
  return lax.gather(v16, idx, dnums, (1,),
                    mode=lax.GatherScatterMode.PROMISE_IN_BOUNDS)


def _make_spmm():
  e_t = E_PAD // NTILES
  nch = e_t // C
  npair = nch // 2
  r_t = e_t // SUB
  nz = N_PAD // NTILES
  zc = 448
  nzrep = nz // zc

  mesh = plsc.VectorSubcoreMesh(
      core_axis_name="c", subcore_axis_name="s",
      num_cores=NCORES, num_subcores=NTILES)

  def body(cols_ref, rows_ref, vals_ref, table_ref, out_ref,
           acc, idx0, idx1, row0, row1, val0, val1, gath0, gath1,
           isem0, isem1, ssem, gs0, gs1, gs2, gs3):
    c = lax.axis_index("c")
    s = lax.axis_index("s")
    gsems = [gs0, gs1, gs2, gs3]
    bufs = [(idx0, row0, val0, gath0, isem0),
            (idx1, row1, val1, gath1, isem1)]

    def in_descs(gi, b):
      idx_b, row_b, val_b, _, sem = bufs[b]
      rb = s * r_t + gi * NSUB
      eb = s * e_t + gi * C
      return [
          pltpu.make_async_copy(cols_ref.at[c, pl.ds(rb, NSUB)], idx_b, sem),
          pltpu.make_async_copy(rows_ref.at[pl.ds(rb, NSUB)], row_b, sem),
          pltpu.make_async_copy(vals_ref.at[pl.ds(eb, C)], val_b, sem),
      ]

    def gath_desc(b, j):
      idx_b, _, _, gath_b, _ = bufs[b]
      return pltpu.make_async_copy(table_ref.at[idx_b.at[j]],
                                   gath_b.at[pl.ds(j * SUB, SUB)], gsems[j])

    def scat_desc(b, j):
      _, row_b, _, gath_b, _ = bufs[b]
      return pltpu.make_async_copy(gath_b.at[pl.ds(j * SUB, SUB)],
                                   acc.at[row_b.at[j]], ssem)

    def scale(b, j):
      _, _, val_b, gath_b, _ = bufs[b]

      def grp(gg, carry):
        base = j * SUB + gg * 16
        v16 = val_b[pl.ds(base, 16)]
        for k in range(16):
          vv = _lane_splat(v16, k)
          gath_b[base + k] = gath_b[base + k] * vv
        return carry
      lax.fori_loop(0, SUB // 16, grp, 0)

    def process(b):
      for j in range(NSUB):
        gath_desc(b, j).wait()
        scale(b, j)
        scat_desc(b, j).start(add=True)

    def zb(j, carry):
      gath0[j] = jnp.zeros((H,), jnp.float32)
      return carry
    lax.fori_loop(0, zc, zb, 0)
    for r in range(nzrep):
      pltpu.sync_copy(gath0.at[pl.ds(0, zc)],
                      acc.at[pl.ds(s * nz + r * zc, zc)])
    plsc.subcore_barrier()

    for d in in_descs(0, 0):
      d.start()
    for d in in_descs(0, 0):
      d.wait()
    for j in range(NSUB):
      gath_desc(0, j).start()

    def pair(p, carry):
      ga = 2 * p
      @pl.when(p > 0)
      def _():
        for j in range(NSUB):
          scat_desc(1, j).wait()
      for d in in_descs(ga + 1, 1):
        d.start()
      process(0)
      for d in in_descs(ga + 1, 1):
        d.wait()
      for j in range(NSUB):
        gath_desc(1, j).start()
      for j in range(NSUB):
        scat_desc(0, j).wait()
      @pl.when(p + 1 < npair)
      def _():
        for d in in_descs(ga + 2, 0):
          d.start()
      process(1)
      @pl.when(p + 1 < npair)
      def _():
        for d in in_descs(ga + 2, 0):
          d.wait()
        for j in range(NSUB):
          gath_desc(0, j).start()
      return carry
    lax.fori_loop(0, npair, pair, 0)

    for j in range(NSUB):
      scat_desc(1, j).wait()

    plsc.subcore_barrier()
    pltpu.sync_copy(acc.at[pl.ds(s * nz, nz)],
                    out_ref.at[pl.ds(c * N_PAD + s * nz, nz)])

  return pl.kernel(
      body,
      out_type=jax.ShapeDtypeStruct((2 * N_PAD, H), jnp.float32),
      mesh=mesh,
      compiler_params=pltpu.CompilerParams(use_tc_tiling_on_sc=False),
      scratch_types=[
          pltpu.VMEM_SHARED((N_PAD, H), jnp.float32),
          pltpu.VMEM((NSUB, SUB), jnp.int32),
          pltpu.VMEM((NSUB, SUB), jnp.int32),
          pltpu.VMEM((NSUB, SUB), jnp.int32),
          pltpu.VMEM((NSUB, SUB), jnp.int32),
          pltpu.VMEM((C,), jnp.float32),
          pltpu.VMEM((C,), jnp.float32),
          pltpu.VMEM((C, H), jnp.float32),
          pltpu.VMEM((C, H), jnp.float32),
          pltpu.SemaphoreType.DMA,
          pltpu.SemaphoreType.DMA,
          pltpu.SemaphoreType.DMA,
          pltpu.SemaphoreType.DMA,
          pltpu.SemaphoreType.DMA,
          pltpu.SemaphoreType.DMA,
          pltpu.SemaphoreType.DMA,
      ],
  )


_BM = 2000


def _mm_body(s_ref, w_ref, hstd_ref, hsp_ref):
  w = w_ref[...]
  s0 = s_ref[0]
  s1 = s_ref[1]
  x = (jnp.dot(s0, w[:H, :], preferred_element_type=jnp.float32) +
       jnp.dot(s1, w[H:, :], preferred_element_type=jnp.float32))
  h = jnp.maximum(x, 0.0)
  hstd_ref[...] = h
  hsp_ref[0] = h[:, :H]
  hsp_ref[1] = h[:, H:]


def _matmul_relu(s2, w):
  grid = N // _BM
  return pl.pallas_call(
      _mm_body,
      grid=(grid,),
      in_specs=[
          pl.BlockSpec((2, _BM, H), lambda i: (0, i, 0)),
          pl.BlockSpec((D, D), lambda i: (0, 0)),
      ],
      out_specs=[
          pl.BlockSpec((_BM, D), lambda i: (i, 0)),
          pl.BlockSpec((2, _BM, H), lambda i: (0, i, 0)),
      ],
      out_shape=[
          jax.ShapeDtypeStruct((N, D), jnp.float32),
          jax.ShapeDtypeStruct((2, N_PAD, H), jnp.float32),
      ],
  )(s2, w)


def _pool_body(e0_ref, e1_ref, e2_ref, w_ref, b_ref, out_ref):
  w = w_ref[...]
  b = b_ref[0, 0]
  e0 = e0_ref[...]
  e1 = e1_ref[...]
  e2 = e2_ref[...]
  a0 = jax.nn.sigmoid(jnp.sum(e0 * w, axis=1, keepdims=True) + b)
  a1 = jax.nn.sigmoid(jnp.sum(e1 * w, axis=1, keepdims=True) + b)
  a2 = jax.nn.sigmoid(jnp.sum(e2 * w, axis=1, keepdims=True) + b)
  m = jnp.maximum(jnp.maximum(a0, a1), a2)
  x0 = jnp.exp(a0 - m)
  x1 = jnp.exp(a1 - m)
  x2 = jnp.exp(a2 - m)
  inv = 1.0 / (x0 + x1 + x2)
  out_ref[...] = (e0 * x0 + e1 * x1 + e2 * x2) * inv


def _pool(e0, e1, e2, w_row, b11):
  grid = N // _BM
  blk = pl.BlockSpec((_BM, D), lambda i: (i, 0))
  return pl.pallas_call(
      _pool_body,
      grid=(grid,),
      in_specs=[blk, blk, blk,
                pl.BlockSpec((1, D), lambda i: (0, 0)),
                pl.BlockSpec((1, 1), lambda i: (0, 0))],
      out_specs=blk,
      out_shape=jax.ShapeDtypeStruct((N, D), jnp.float32),
  )(e0, e1, e2, w_row, b11)


def _split_pad(x):
  x2 = x.reshape(N, 2, H).transpose(1, 0, 2)
  x2 = jnp.pad(x2, ((0, 0), (0, N_PAD - N), (0, 0)))
  return x2.reshape(2 * N_PAD, H)


def _branch(indices, values, emb0, w, attn_w, attn_b, spmm):
  rows = jnp.pad(indices[0], (0, E_PAD - E))
  cols = jnp.pad(indices[1], (0, E_PAD - E))
  vals = jnp.pad(values, (0, E_PAD - E))
  cols3 = jnp.stack([cols, cols + N_PAD]).reshape(2, E_PAD // SUB, SUB)
  rows3 = rows.reshape(E_PAD // SUB, SUB)

  s0 = spmm(cols3, rows3, vals, _split_pad(emb0))
  h1_std, h1_sp = _matmul_relu(s0.reshape(2, N_PAD, H), w)
  s1 = spmm(cols3, rows3, vals, h1_sp.reshape(2 * N_PAD, H))
  h2_std, _ = _matmul_relu(s1.reshape(2, N_PAD, H), w)
  return _pool(emb0, h1_std, h2_std,
               attn_w.reshape(1, D), attn_b.reshape(1, 1))


def kernel(adj_u1_indices, adj_u1_values, adj_i1_indices, adj_i1_values,
           user_emb, item_emb, W_u, W_i,
           attn_u_w, attn_u_b, attn_i_w, attn_i_b):
  spmm = _make_spmm()
  u_out = _branch(adj_u1_indices, adj_u1_values, user_emb, W_u,
                  attn_u_w, attn_u_b, spmm)
  i_out = _branch(adj_i1_indices, adj_i1_values, item_emb, W_i,
                  attn_i_w, attn_i_b, spmm)
  return (u_out, i_out)


# speedup vs baseline: 13.5856x; 1.0592x over previous
import jax
import jax.numpy as jnp
from jax import lax
from jax.experimental import pallas as pl
from jax.experimental.pallas import tpu as pltpu
from jax.experimental.pallas import tpu_sc as plsc

N = 100000
D = 32
H = 16
E = 1600000
NCORES = 2
NTILES = 16
SUB = 128
NSUB = 4
C = SUB * NSUB
N_PAD = 100352
E_PAD = 1605632


def _lane_splat(v16, k):
  idx = jnp.full((16, 1), k, jnp.int32)
  dnums = lax.GatherDimensionNumbers(
      offset_dims=(), collapsed_slice_dims=(0,), start_index_map=(0,))
  return lax.gather(v16, idx, dnums, (1,),
                    mode=lax.GatherScatterMode.PROMISE_IN_BOUNDS)


def _make_spmm():
  e_t = E_PAD // NTILES
  nch = e_t // C
  npair = nch // 2
  r_t = e_t // SUB
  nz = N_PAD // NTILES
  zc = 448
  nzrep = nz // zc

  mesh = plsc.VectorSubcoreMesh(
      core_axis_name="c", subcore_axis_name="s",
      num_cores=NCORES, num_subcores=NTILES)

  def body(cols_ref, rows_ref, vals_ref, table_ref, out_ref,
           acc, idx0, idx1, row0, row1, val0, val1, gath0, gath1,
           isem0, isem1, ssem, gs0, gs1, gs2, gs3):
    c = lax.axis_index("c")
    s = lax.axis_index("s")
    gsems = [gs0, gs1, gs2, gs3]
    bufs = [(idx0, row0, val0, gath0, isem0),
            (idx1, row1, val1, gath1, isem1)]

    def in_descs(gi, b):
      idx_b, row_b, val_b, _, sem = bufs[b]
      rb = s * r_t + gi * NSUB
      eb = s * e_t + gi * C
      return [
          pltpu.make_async_copy(cols_ref.at[c, pl.ds(rb, NSUB)], idx_b, sem),
          pltpu.make_async_copy(rows_ref.at[pl.ds(rb, NSUB)], row_b, sem),
          pltpu.make_async_copy(vals_ref.at[pl.ds(eb, C)], val_b, sem),
      ]

    def gath_desc(b, j):
      idx_b, _, _, gath_b, _ = bufs[b]
      return pltpu.make_async_copy(table_ref.at[idx_b.at[j]],
                                   gath_b.at[pl.ds(j * SUB, SUB)], gsems[j])

    def scat_desc(b, j):
      _, row_b, _, gath_b, _ = bufs[b]
      return pltpu.make_async_copy(gath_b.at[pl.ds(j * SUB, SUB)],
                                   acc.at[row_b.at[j]], ssem)

    def scale(b, j):
      _, _, val_b, gath_b, _ = bufs[b]

      @plsc.parallel_loop(0, SUB // 16, 1, unroll=2)
      def grp(gg):
        base = j * SUB + gg * 16
        v16 = val_b[pl.ds(base, 16)]
        for k in range(16):
          vv = _lane_splat(v16, k)
          gath_b[base + k] = gath_b[base + k] * vv

    def process(b, js):
      for j in js:
        gath_desc(b, j).wait()
        scale(b, j)
        scat_desc(b, j).start(add=True)

    def zb(j, carry):
      gath0[j] = jnp.zeros((H,), jnp.float32)
      return carry
    lax.fori_loop(0, zc, zb, 0)
    for r in range(nzrep):
      pltpu.sync_copy(gath0.at[pl.ds(0, zc)],
                      acc.at[pl.ds(s * nz + r * zc, zc)])
    plsc.subcore_barrier()

    for d in in_descs(0, 0):
      d.start()
    for d in in_descs(0, 0):
      d.wait()
    for j in range(NSUB):
      gath_desc(0, j).start()

    half0 = tuple(range(NSUB // 2))
    half1 = tuple(range(NSUB // 2, NSUB))

    def pair(p, carry):
      ga = 2 * p
      @pl.when(p > 0)
      def _():
        for j in range(NSUB):
          scat_desc(1, j).wait()
      for d in in_descs(ga + 1, 1):
        d.start()
      process(0, half0)
      for d in in_descs(ga + 1, 1):
        d.wait()
      for j in range(NSUB):
        gath_desc(1, j).start()
      process(0, half1)
      for j in range(NSUB):
        scat_desc(0, j).wait()
      @pl.when(p + 1 < npair)
      def _():
        for d in in_descs(ga + 2, 0):
          d.start()
      process(1, half0)
      @pl.when(p + 1 < npair)
      def _():
        for d in in_descs(ga + 2, 0):
          d.wait()
        for j in range(NSUB):
          gath_desc(0, j).start()
      process(1, half1)
      return carry
    lax.fori_loop(0, npair, pair, 0)

    for j in range(NSUB):
      scat_desc(1, j).wait()

    plsc.subcore_barrier()
    pltpu.sync_copy(acc.at[pl.ds(s * nz, nz)],
                    out_ref.at[pl.ds(c * N_PAD + s * nz, nz)])

  return pl.kernel(
      body,
      out_type=jax.ShapeDtypeStruct((2 * N_PAD, H), jnp.float32),
      mesh=mesh,
      compiler_params=pltpu.CompilerParams(use_tc_tiling_on_sc=False),
      scratch_types=[
          pltpu.VMEM_SHARED((N_PAD, H), jnp.float32),
          pltpu.VMEM((NSUB, SUB), jnp.int32),
          pltpu.VMEM((NSUB, SUB), jnp.int32),
          pltpu.VMEM((NSUB, SUB), jnp.int32),
          pltpu.VMEM((NSUB, SUB), jnp.int32),
          pltpu.VMEM((C,), jnp.float32),
          pltpu.VMEM((C,), jnp.float32),
          pltpu.VMEM((C, H), jnp.float32),
          pltpu.VMEM((C, H), jnp.float32),
          pltpu.SemaphoreType.DMA,
          pltpu.SemaphoreType.DMA,
          pltpu.SemaphoreType.DMA,
          pltpu.SemaphoreType.DMA,
          pltpu.SemaphoreType.DMA,
          pltpu.SemaphoreType.DMA,
          pltpu.SemaphoreType.DMA,
      ],
  )


_BM = 2000


def _mm_body(s_ref, w_ref, hstd_ref, hsp_ref):
  w = w_ref[...]
  s0 = s_ref[0]
  s1 = s_ref[1]
  x = (jnp.dot(s0, w[:H, :], preferred_element_type=jnp.float32) +
       jnp.dot(s1, w[H:, :], preferred_element_type=jnp.float32))
  h = jnp.maximum(x, 0.0)
  hstd_ref[...] = h
  hsp_ref[0] = h[:, :H]
  hsp_ref[1] = h[:, H:]


def _matmul_relu(s2, w):
  grid = N // _BM
  return pl.pallas_call(
      _mm_body,
      grid=(grid,),
      in_specs=[
          pl.BlockSpec((2, _BM, H), lambda i: (0, i, 0)),
          pl.BlockSpec((D, D), lambda i: (0, 0)),
      ],
      out_specs=[
          pl.BlockSpec((_BM, D), lambda i: (i, 0)),
          pl.BlockSpec((2, _BM, H), lambda i: (0, i, 0)),
      ],
      out_shape=[
          jax.ShapeDtypeStruct((N, D), jnp.float32),
          jax.ShapeDtypeStruct((2, N_PAD, H), jnp.float32),
      ],
  )(s2, w)


def _pool_body(e0_ref, e1_ref, e2_ref, w_ref, b_ref, out_ref):
  w = w_ref[...]
  b = b_ref[0, 0]
  e0 = e0_ref[...]
  e1 = e1_ref[...]
  e2 = e2_ref[...]
  a0 = jax.nn.sigmoid(jnp.sum(e0 * w, axis=1, keepdims=True) + b)
  a1 = jax.nn.sigmoid(jnp.sum(e1 * w, axis=1, keepdims=True) + b)
  a2 = jax.nn.sigmoid(jnp.sum(e2 * w, axis=1, keepdims=True) + b)
  m = jnp.maximum(jnp.maximum(a0, a1), a2)
  x0 = jnp.exp(a0 - m)
  x1 = jnp.exp(a1 - m)
  x2 = jnp.exp(a2 - m)
  inv = 1.0 / (x0 + x1 + x2)
  out_ref[...] = (e0 * x0 + e1 * x1 + e2 * x2) * inv


def _pool(e0, e1, e2, w_row, b11):
  grid = N // _BM
  blk = pl.BlockSpec((_BM, D), lambda i: (i, 0))
  return pl.pallas_call(
      _pool_body,
      grid=(grid,),
      in_specs=[blk, blk, blk,
                pl.BlockSpec((1, D), lambda i: (0, 0)),
                pl.BlockSpec((1, 1), lambda i: (0, 0))],
      out_specs=blk,
      out_shape=jax.ShapeDtypeStruct((N, D), jnp.float32),
  )(e0, e1, e2, w_row, b11)


def _split_pad(x):
  x2 = x.reshape(N, 2, H).transpose(1, 0, 2)
  x2 = jnp.pad(x2, ((0, 0), (0, N_PAD - N), (0, 0)))
  return x2.reshape(2 * N_PAD, H)


def _branch(indices, values, emb0, w, attn_w, attn_b, spmm):
  rows = jnp.pad(indices[0], (0, E_PAD - E))
  cols = jnp.pad(indices[1], (0, E_PAD - E))
  vals = jnp.pad(values, (0, E_PAD - E))
  cols3 = jnp.stack([cols, cols + N_PAD]).reshape(2, E_PAD // SUB, SUB)
  rows3 = rows.reshape(E_PAD // SUB, SUB)

  s0 = spmm(cols3, rows3, vals, _split_pad(emb0))
  h1_std, h1_sp = _matmul_relu(s0.reshape(2, N_PAD, H), w)
  s1 = spmm(cols3, rows3, vals, h1_sp.reshape(2 * N_PAD, H))
  h2_std, _ = _matmul_relu(s1.reshape(2, N_PAD, H), w)
  return _pool(emb0, h1_std, h2_std,
               attn_w.reshape(1, D), attn_b.reshape(1, 1))


def kernel(adj_u1_indices, adj_u1_values, adj_i1_indices, adj_i1_values,
           user_emb, item_emb, W_u, W_i,
           attn_u_w, attn_u_b, attn_i_w, attn_i_b):
  spmm = _make_spmm()
  u_out = _branch(adj_u1_indices, adj_u1_values, user_emb, W_u,
                  attn_u_w, attn_u_b, spmm)
  i_out = _branch(adj_i1_indices, adj_i1_values, item_emb, W_i,
                  attn_i_w, attn_i_b, spmm)
  return (u_out, i_out)
